# Initial kernel scaffold; baseline (speedup 1.0000x reference)
import jax
import jax.numpy as jnp
from jax.experimental import pallas as pl


def kernel(nodes, neighbors, attention_scores, W, bias):
    raise NotImplementedError("write your pallas kernel here")


# speedup vs baseline: 1.1819x; 1.1819x over previous
import functools

import jax
import jax.numpy as jnp
from jax.experimental import pallas as pl

N = 10000
DEG = 32
D_IN = 128
D_OUT = 128
BN = 400


def _grn_body(neigh_ref, att_ref, w_ref, b_ref, out_ref):
    neigh = neigh_ref[...]
    att = att_ref[...]
    agg = jnp.sum(neigh * att[:, :, None], axis=1)
    proj = jax.lax.dot_general(
        agg, w_ref[...], (((1,), (1,)), ((), ())),
        preferred_element_type=jnp.float32)
    out = proj + b_ref[...]
    out_ref[...] = jnp.where(out > 0, out, jnp.exp(jnp.minimum(out, 0.0)) - 1.0)


@jax.jit
def kernel(nodes, neighbors, attention_scores, W, bias):
    del nodes
    bias2d = bias.reshape(1, D_OUT)
    grid = N // BN
    return pl.pallas_call(
        _grn_body,
        grid=(grid,),
        in_specs=[
            pl.BlockSpec((BN, DEG, D_IN), lambda i: (i, 0, 0)),
            pl.BlockSpec((BN, DEG), lambda i: (i, 0)),
            pl.BlockSpec((D_OUT, D_IN), lambda i: (0, 0)),
            pl.BlockSpec((1, D_OUT), lambda i: (0, 0)),
        ],
        out_specs=pl.BlockSpec((BN, D_OUT), lambda i: (i, 0)),
        out_shape=jax.ShapeDtypeStruct((N, D_OUT), jnp.float32),
    )(neighbors, attention_scores, W, bias2d)
